# initial kernel scaffold (unmeasured)
import jax
import jax.numpy as jnp
from jax import lax
from jax.experimental import pallas as pl
from jax.experimental.pallas import tpu as pltpu

M = 4096
N = 4096
KC = 1024


def kernel(dy, W):
    my_x = lax.axis_index("x")
    my_y = lax.axis_index("y")
    my_z = lax.axis_index("z")
    idx8 = my_x * 4 + my_z
    dy_c = lax.dynamic_slice(dy, (0, idx8 * KC), (M, KC))
    w_c = lax.dynamic_slice(W, (0, idx8 * KC), (N, KC))
    p = lax.dot_general(
        dy_c, w_c, (((1,), (1,)), ((), ())),
        preferred_element_type=jnp.float32,
    )

    def body(p_ref, o_ref, recv_ref, send_sems, recv_sems,
             r0, r1, r2, r3, r4, r5, r6, r7, copy_sem):
        ready = [r0, r1, r2, r3, r4, r5, r6, r7]
        x = lax.axis_index("x")
        y = lax.axis_index("y")
        z = lax.axis_index("z")
        zb0 = z % 2
        zb1 = z // 2

        rowX = x * 2048
        rowY = rowX + y * 1024
        rowZ1 = rowY + zb0 * 512
        rowR = rowZ1 + zb1 * 256

        px = (1 - x, y, z)
        py = (x, 1 - y, z)
        pz1 = (x, y, z + 1 - 2 * zb0)
        pz2 = (x, y, z + 2 - 4 * zb1)

        bar = pltpu.get_barrier_semaphore()
        for pid in (px, py, pz1, pz2):
            pl.semaphore_signal(
                bar, inc=1, device_id=pid,
                device_id_type=pl.DeviceIdType.MESH,
            )
        pl.semaphore_wait(bar, 4)

        cp = pltpu.make_async_copy(
            p_ref.at[pl.ds(rowX, 2048), :],
            o_ref.at[pl.ds(rowX, 2048), :],
            copy_sem,
        )
        cp.start()

        def xchg(s, partner, src, dst):
            pl.semaphore_signal(
                ready[s], inc=1, device_id=partner,
                device_id_type=pl.DeviceIdType.MESH,
            )
            pl.semaphore_wait(ready[s], 1)
            rd = pltpu.make_async_remote_copy(
                src_ref=src, dst_ref=dst,
                send_sem=send_sems.at[s], recv_sem=recv_sems.at[s],
                device_id=partner, device_id_type=pl.DeviceIdType.MESH,
            )
            rd.start()
            rd.wait()

        xchg(0, px,
             p_ref.at[pl.ds((1 - x) * 2048, 2048), :],
             recv_ref.at[pl.ds(0, 2048), :])
        cp.wait()
        o_ref[pl.ds(rowX, 2048), :] = (
            o_ref[pl.ds(rowX, 2048), :] + recv_ref[0:2048, :])

        xchg(1, py,
             o_ref.at[pl.ds(rowX + (1 - y) * 1024, 1024), :],
             recv_ref.at[pl.ds(0, 1024), :])
        o_ref[pl.ds(rowY, 1024), :] = (
            o_ref[pl.ds(rowY, 1024), :] + recv_ref[0:1024, :])

        xchg(2, pz1,
             o_ref.at[pl.ds(rowY + (1 - zb0) * 512, 512), :],
             recv_ref.at[pl.ds(0, 512), :])
        o_ref[pl.ds(rowZ1, 512), :] = (
            o_ref[pl.ds(rowZ1, 512), :] + recv_ref[0:512, :])

        xchg(3, pz2,
             o_ref.at[pl.ds(rowZ1 + (1 - zb1) * 256, 256), :],
             recv_ref.at[pl.ds(0, 256), :])
        o_ref[pl.ds(rowR, 256), :] = (
            o_ref[pl.ds(rowR, 256), :] + recv_ref[0:256, :])

        xchg(4, pz2, o_ref.at[pl.ds(rowR, 256), :],
             o_ref.at[pl.ds(rowR, 256), :])
        xchg(5, pz1, o_ref.at[pl.ds(rowZ1, 512), :],
             o_ref.at[pl.ds(rowZ1, 512), :])
        xchg(6, py, o_ref.at[pl.ds(rowY, 1024), :],
             o_ref.at[pl.ds(rowY, 1024), :])
        xchg(7, px, o_ref.at[pl.ds(rowX, 2048), :],
             o_ref.at[pl.ds(rowX, 2048), :])

    return pl.pallas_call(
        body,
        out_shape=jax.ShapeDtypeStruct((M, N), jnp.float32),
        in_specs=[pl.BlockSpec(memory_space=pltpu.ANY)],
        out_specs=pl.BlockSpec(memory_space=pltpu.VMEM),
        scratch_shapes=[
            pltpu.VMEM((2048, N), jnp.float32),
            pltpu.SemaphoreType.DMA((8,)),
            pltpu.SemaphoreType.DMA((8,)),
        ] + [pltpu.SemaphoreType.REGULAR] * 8
          + [pltpu.SemaphoreType.DMA],
        compiler_params=pltpu.CompilerParams(
            collective_id=0,
            vmem_limit_bytes=128 * 1024 * 1024,
        ),
    )(p)


# baseline (device time: 1587700 ns/iter reference)
import jax
import jax.numpy as jnp
from jax import lax
from jax.experimental import pallas as pl
from jax.experimental.pallas import tpu as pltpu

M = 4096
N = 4096
KC = 1024


def kernel(dy, W):
    my_x = lax.axis_index("x")
    my_z = lax.axis_index("z")
    idx8 = my_x * 4 + my_z
    dy_c = lax.dynamic_slice(dy, (0, idx8 * KC), (M, KC))
    w_c = lax.dynamic_slice(W, (0, idx8 * KC), (N, KC))
    p = lax.dot_general(
        dy_c, w_c, (((1,), (1,)), ((), ())),
        preferred_element_type=jnp.float32,
    )

    def body(p_ref, o_ref, acc, recv, send_sems, recv_sems,
             r0, r1, r2, r3, r4, r5, r6, r7, r8, lsems):
        ready = [r0, r1, r2, r3, r4, r5, r6, r7, r8]
        x = lax.axis_index("x")
        y = lax.axis_index("y")
        z = lax.axis_index("z")
        zb0 = z % 2
        zb1 = z // 2

        rowX = x * 2048
        rowY = rowX + y * 1024
        rowZ1 = rowY + zb0 * 512
        rowR = rowZ1 + zb1 * 256
        a1 = y * 1024
        a2 = a1 + zb0 * 512
        a3 = a2 + zb1 * 256

        px = (1 - x, y, z)
        py = (x, 1 - y, z)
        pz1 = (x, y, z + 1 - 2 * zb0)
        pz2 = (x, y, z + 2 - 4 * zb1)

        bar = pltpu.get_barrier_semaphore()
        for pid in (px, py, pz1, pz2):
            pl.semaphore_signal(
                bar, inc=1, device_id=pid,
                device_id_type=pl.DeviceIdType.MESH,
            )
        pl.semaphore_wait(bar, 4)

        cps = []
        for c in range(2):
            cp = pltpu.make_async_copy(
                p_ref.at[pl.ds(rowX + c * 1024, 1024), :],
                acc.at[pl.ds(c * 1024, 1024), :],
                lsems.at[c],
            )
            cp.start()
            cps.append(cp)

        def xchg(t, partner, src, dst):
            pl.semaphore_signal(
                ready[t], inc=1, device_id=partner,
                device_id_type=pl.DeviceIdType.MESH,
            )
            pl.semaphore_wait(ready[t], 1)
            rd = pltpu.make_async_remote_copy(
                src_ref=src, dst_ref=dst,
                send_sem=send_sems.at[t], recv_sem=recv_sems.at[t],
                device_id=partner, device_id_type=pl.DeviceIdType.MESH,
            )
            rd.start()
            rd.wait()

        for c in range(2):
            xchg(c, px,
                 p_ref.at[pl.ds((1 - x) * 2048 + c * 1024, 1024), :],
                 recv.at[pl.ds(0, 1024), :])
            cps[c].wait()
            acc[pl.ds(c * 1024, 1024), :] = (
                acc[pl.ds(c * 1024, 1024), :] + recv[0:1024, :])

        xchg(2, py, acc.at[pl.ds((1 - y) * 1024, 1024), :],
             recv.at[pl.ds(0, 1024), :])
        acc[pl.ds(a1, 1024), :] = acc[pl.ds(a1, 1024), :] + recv[0:1024, :]

        xchg(3, pz1, acc.at[pl.ds(a1 + (1 - zb0) * 512, 512), :],
             recv.at[pl.ds(0, 512), :])
        acc[pl.ds(a2, 512), :] = acc[pl.ds(a2, 512), :] + recv[0:512, :]

        xchg(4, pz2, acc.at[pl.ds(a2 + (1 - zb1) * 256, 256), :],
             recv.at[pl.ds(0, 256), :])
        acc[pl.ds(a3, 256), :] = acc[pl.ds(a3, 256), :] + recv[0:256, :]

        cpo = pltpu.make_async_copy(
            acc.at[pl.ds(a3, 256), :],
            o_ref.at[pl.ds(rowR, 256), :],
            lsems.at[2],
        )
        cpo.start()
        cpo.wait()

        xchg(5, pz2, o_ref.at[pl.ds(rowR, 256), :],
             o_ref.at[pl.ds(rowR, 256), :])
        xchg(6, pz1, o_ref.at[pl.ds(rowZ1, 512), :],
             o_ref.at[pl.ds(rowZ1, 512), :])
        xchg(7, py, o_ref.at[pl.ds(rowY, 1024), :],
             o_ref.at[pl.ds(rowY, 1024), :])
        xchg(8, px, o_ref.at[pl.ds(rowX, 2048), :],
             o_ref.at[pl.ds(rowX, 2048), :])

    return pl.pallas_call(
        body,
        out_shape=jax.ShapeDtypeStruct((M, N), jnp.float32),
        in_specs=[pl.BlockSpec(memory_space=pl.ANY)],
        out_specs=pl.BlockSpec(memory_space=pl.ANY),
        scratch_shapes=[
            pltpu.VMEM((2048, N), jnp.float32),
            pltpu.VMEM((1024, N), jnp.float32),
            pltpu.SemaphoreType.DMA((9,)),
            pltpu.SemaphoreType.DMA((9,)),
        ] + [pltpu.SemaphoreType.REGULAR] * 9
          + [pltpu.SemaphoreType.DMA((3,))],
        compiler_params=pltpu.CompilerParams(
            collective_id=0,
            vmem_limit_bytes=56 * 1024 * 1024,
        ),
    )(p)


# device time: 953060 ns/iter; 1.6659x vs baseline; 1.6659x over previous
import jax
import jax.numpy as jnp
from jax import lax
from jax.experimental import pallas as pl
from jax.experimental.pallas import tpu as pltpu

M = 4096
N = 4096
KC = 1024

STREAM_ORDERS = (("x", "y", "z1", "z2"), ("y", "z1", "z2", "x"))
N_CHUNKS = 1
HALF = (2048, 1024, 512, 256)


def kernel(dy, W):
    my_x = lax.axis_index("x")
    my_z = lax.axis_index("z")
    idx8 = my_x * 4 + my_z
    dy_c = lax.dynamic_slice(dy, (0, idx8 * KC), (M, KC))
    w_c = lax.dynamic_slice(W, (0, idx8 * KC), (N, KC))
    p = lax.dot_general(
        dy_c, w_c, (((1,), (1,)), ((), ())),
        preferred_element_type=jnp.float32,
    )

    n_streams = len(STREAM_ORDERS)
    n_flows = n_streams * N_CHUNKS
    n_tids = 9 * n_flows

    def body(p_ref, o_ref, acc, recv, send_sems, recv_sems, ready, lsems):
        x = lax.axis_index("x")
        y = lax.axis_index("y")
        z = lax.axis_index("z")
        zb0 = z % 2
        zb1 = z // 2

        B = {"x": x, "y": y, "z1": zb0, "z2": zb1}
        P = {
            "x": (1 - x, y, z),
            "y": (x, 1 - y, z),
            "z1": (x, y, z + 1 - 2 * zb0),
            "z2": (x, y, z + 2 - 4 * zb1),
        }

        bar = pltpu.get_barrier_semaphore()
        for pid in P.values():
            pl.semaphore_signal(
                bar, inc=1, device_id=pid,
                device_id_type=pl.DeviceIdType.MESH,
            )
        pl.semaphore_wait(bar, 4)

        flows = []
        cw = N // n_flows
        for si, order in enumerate(STREAM_ORDERS):
            for c in range(N_CHUNKS):
                f = {
                    "order": order,
                    "c0": (si * N_CHUNKS + c) * cw,
                    "fi": si * N_CHUNKS + c,
                }
                gb = 0
                gbs = [0]
                send_g = []
                for j, d in enumerate(order):
                    send_g.append(gb + (1 - B[d]) * HALF[j])
                    gb = gb + B[d] * HALF[j]
                    gbs.append(gb)
                f["gbs"] = gbs
                f["send_g"] = send_g
                flows.append(f)

        pend = {}

        def credit_and_start(tid, partner, src, dst):
            pl.semaphore_signal(
                ready.at[tid], inc=1, device_id=partner,
                device_id_type=pl.DeviceIdType.MESH,
            )
            pl.semaphore_wait(ready.at[tid], 1)
            rd = pltpu.make_async_remote_copy(
                src_ref=src, dst_ref=dst,
                send_sem=send_sems.at[tid], recv_sem=recv_sems.at[tid],
                device_id=partner, device_id_type=pl.DeviceIdType.MESH,
            )
            rd.start()
            return rd

        def issue(f, w):
            fi, c0, gbs, order = f["fi"], f["c0"], f["gbs"], f["order"]
            cols = pl.ds(c0, cw)
            tid = 9 * fi + w
            if w in (0, 1):
                sub = w
                if w == 0:
                    f["cps"] = []
                    for s in range(2):
                        cp = pltpu.make_async_copy(
                            p_ref.at[pl.ds(gbs[1] + s * 1024, 1024), cols],
                            acc.at[pl.ds(s * 1024, 1024), cols],
                            lsems.at[3 * fi + s],
                        )
                        cp.start()
                        f["cps"].append(cp)
                rd = credit_and_start(
                    tid, P[order[0]],
                    p_ref.at[pl.ds(f["send_g"][0] + sub * 1024, 1024), cols],
                    recv.at[pl.ds(0, 1024), cols],
                )

                def fin(rd=rd, sub=sub, cp=f["cps"][sub], c0=c0):
                    rd.wait()
                    cp.wait()
                    r = pl.ds(sub * 1024, 1024)
                    acc[r, pl.ds(c0, cw)] = (
                        acc[r, pl.ds(c0, cw)] + recv[0:1024, c0:c0 + cw])
                pend[(fi, w)] = fin
            elif w in (2, 3, 4):
                j = w - 1
                n = HALF[j]
                cb_src = f["send_g"][j] - gbs[1]
                cb_dst = gbs[j + 1] - gbs[1]
                rd = credit_and_start(
                    tid, P[order[j]],
                    acc.at[pl.ds(cb_src, n), cols],
                    recv.at[pl.ds(0, n), cols],
                )

                def fin(rd=rd, n=n, cb_dst=cb_dst, c0=c0):
                    rd.wait()
                    r = pl.ds(cb_dst, n)
                    acc[r, pl.ds(c0, cw)] = (
                        acc[r, pl.ds(c0, cw)] + recv[0:n, c0:c0 + cw])
                pend[(fi, w)] = fin
            elif w == 5:
                cp = pltpu.make_async_copy(
                    acc.at[pl.ds(gbs[4] - gbs[1], 256), cols],
                    o_ref.at[pl.ds(gbs[4], 256), cols],
                    lsems.at[3 * fi + 2],
                )
                cp.start()
                rd = credit_and_start(
                    tid, P[order[3]],
                    acc.at[pl.ds(gbs[4] - gbs[1], 256), cols],
                    o_ref.at[pl.ds(gbs[4], 256), cols],
                )

                def fin(rd=rd, cp=cp):
                    rd.wait()
                    cp.wait()
                pend[(fi, w)] = fin
            else:
                k = w - 5
                jlev = 4 - k
                n = HALF[jlev - 1]
                rd = credit_and_start(
                    tid, P[order[jlev - 1]],
                    o_ref.at[pl.ds(gbs[jlev], n), cols],
                    o_ref.at[pl.ds(gbs[jlev], n), cols],
                )

                def fin(rd=rd):
                    rd.wait()
                pend[(fi, w)] = fin

        for w in range(9):
            for f in flows:
                if w > 0:
                    pend.pop((f["fi"], w - 1))()
                issue(f, w)
        for f in flows:
            pend.pop((f["fi"], 8))()

    return pl.pallas_call(
        body,
        out_shape=jax.ShapeDtypeStruct((M, N), jnp.float32),
        in_specs=[pl.BlockSpec(memory_space=pl.ANY)],
        out_specs=pl.BlockSpec(memory_space=pl.ANY),
        scratch_shapes=[
            pltpu.VMEM((2048, N), jnp.float32),
            pltpu.VMEM((1024, N), jnp.float32),
            pltpu.SemaphoreType.DMA((n_tids,)),
            pltpu.SemaphoreType.DMA((n_tids,)),
            pltpu.SemaphoreType.REGULAR((n_tids,)),
            pltpu.SemaphoreType.DMA((3 * n_flows,)),
        ],
        compiler_params=pltpu.CompilerParams(
            collective_id=0,
            vmem_limit_bytes=56 * 1024 * 1024,
        ),
    )(p)


# device time: 910563 ns/iter; 1.7436x vs baseline; 1.0467x over previous
import jax
import jax.numpy as jnp
from jax import lax
from jax.experimental import pallas as pl
from jax.experimental.pallas import tpu as pltpu

M = 4096
N = 4096
KC = 1024

STREAM_ORDERS = (("x", "y", "z1", "z2"), ("y", "z1", "z2", "x"))
N_CHUNKS = 2
HALF = (2048, 1024, 512, 256)


def kernel(dy, W):
    my_x = lax.axis_index("x")
    my_z = lax.axis_index("z")
    idx8 = my_x * 4 + my_z
    dy_c = lax.dynamic_slice(dy, (0, idx8 * KC), (M, KC))
    w_c = lax.dynamic_slice(W, (0, idx8 * KC), (N, KC))
    p = lax.dot_general(
        dy_c, w_c, (((1,), (1,)), ((), ())),
        preferred_element_type=jnp.float32,
    )

    n_streams = len(STREAM_ORDERS)
    n_flows = n_streams * N_CHUNKS
    n_tids = 9 * n_flows

    def body(p_ref, o_ref, acc, recv, send_sems, recv_sems, ready, lsems):
        x = lax.axis_index("x")
        y = lax.axis_index("y")
        z = lax.axis_index("z")
        zb0 = z % 2
        zb1 = z // 2

        B = {"x": x, "y": y, "z1": zb0, "z2": zb1}
        P = {
            "x": (1 - x, y, z),
            "y": (x, 1 - y, z),
            "z1": (x, y, z + 1 - 2 * zb0),
            "z2": (x, y, z + 2 - 4 * zb1),
        }

        bar = pltpu.get_barrier_semaphore()
        for pid in P.values():
            pl.semaphore_signal(
                bar, inc=1, device_id=pid,
                device_id_type=pl.DeviceIdType.MESH,
            )
        pl.semaphore_wait(bar, 4)

        flows = []
        cw = N // n_flows
        for si, order in enumerate(STREAM_ORDERS):
            for c in range(N_CHUNKS):
                f = {
                    "order": order,
                    "c0": (si * N_CHUNKS + c) * cw,
                    "fi": si * N_CHUNKS + c,
                }
                gb = 0
                gbs = [0]
                send_g = []
                for j, d in enumerate(order):
                    send_g.append(gb + (1 - B[d]) * HALF[j])
                    gb = gb + B[d] * HALF[j]
                    gbs.append(gb)
                f["gbs"] = gbs
                f["send_g"] = send_g
                flows.append(f)

        pend = {}

        def credit_and_start(tid, partner, src, dst):
            pl.semaphore_signal(
                ready.at[tid], inc=1, device_id=partner,
                device_id_type=pl.DeviceIdType.MESH,
            )
            pl.semaphore_wait(ready.at[tid], 1)
            rd = pltpu.make_async_remote_copy(
                src_ref=src, dst_ref=dst,
                send_sem=send_sems.at[tid], recv_sem=recv_sems.at[tid],
                device_id=partner, device_id_type=pl.DeviceIdType.MESH,
            )
            rd.start()
            return rd

        def issue(f, w):
            fi, c0, gbs, order = f["fi"], f["c0"], f["gbs"], f["order"]
            cols = pl.ds(c0, cw)
            tid = 9 * fi + w
            if w in (0, 1):
                sub = w
                if w == 0:
                    f["cps"] = []
                    for s in range(2):
                        cp = pltpu.make_async_copy(
                            p_ref.at[pl.ds(gbs[1] + s * 1024, 1024), cols],
                            acc.at[pl.ds(s * 1024, 1024), cols],
                            lsems.at[3 * fi + s],
                        )
                        cp.start()
                        f["cps"].append(cp)
                rd = credit_and_start(
                    tid, P[order[0]],
                    p_ref.at[pl.ds(f["send_g"][0] + sub * 1024, 1024), cols],
                    recv.at[pl.ds(0, 1024), cols],
                )

                def fin(rd=rd, sub=sub, cp=f["cps"][sub], c0=c0):
                    rd.wait()
                    cp.wait()
                    r = pl.ds(sub * 1024, 1024)
                    acc[r, pl.ds(c0, cw)] = (
                        acc[r, pl.ds(c0, cw)] + recv[0:1024, c0:c0 + cw])
                pend[(fi, w)] = fin
            elif w in (2, 3, 4):
                j = w - 1
                n = HALF[j]
                cb_src = f["send_g"][j] - gbs[1]
                cb_dst = gbs[j + 1] - gbs[1]
                rd = credit_and_start(
                    tid, P[order[j]],
                    acc.at[pl.ds(cb_src, n), cols],
                    recv.at[pl.ds(0, n), cols],
                )

                def fin(rd=rd, n=n, cb_dst=cb_dst, c0=c0):
                    rd.wait()
                    r = pl.ds(cb_dst, n)
                    acc[r, pl.ds(c0, cw)] = (
                        acc[r, pl.ds(c0, cw)] + recv[0:n, c0:c0 + cw])
                pend[(fi, w)] = fin
            elif w == 5:
                cp = pltpu.make_async_copy(
                    acc.at[pl.ds(gbs[4] - gbs[1], 256), cols],
                    o_ref.at[pl.ds(gbs[4], 256), cols],
                    lsems.at[3 * fi + 2],
                )
                cp.start()
                rd = credit_and_start(
                    tid, P[order[3]],
                    acc.at[pl.ds(gbs[4] - gbs[1], 256), cols],
                    o_ref.at[pl.ds(gbs[4], 256), cols],
                )

                def fin(rd=rd, cp=cp):
                    rd.wait()
                    cp.wait()
                pend[(fi, w)] = fin
            else:
                k = w - 5
                jlev = 4 - k
                n = HALF[jlev - 1]
                rd = credit_and_start(
                    tid, P[order[jlev - 1]],
                    o_ref.at[pl.ds(gbs[jlev], n), cols],
                    o_ref.at[pl.ds(gbs[jlev], n), cols],
                )

                def fin(rd=rd):
                    rd.wait()
                pend[(fi, w)] = fin

        for w in range(9):
            for f in flows:
                if w > 0:
                    pend.pop((f["fi"], w - 1))()
                issue(f, w)
        for f in flows:
            pend.pop((f["fi"], 8))()

    return pl.pallas_call(
        body,
        out_shape=jax.ShapeDtypeStruct((M, N), jnp.float32),
        in_specs=[pl.BlockSpec(memory_space=pl.ANY)],
        out_specs=pl.BlockSpec(memory_space=pl.ANY),
        scratch_shapes=[
            pltpu.VMEM((2048, N), jnp.float32),
            pltpu.VMEM((1024, N), jnp.float32),
            pltpu.SemaphoreType.DMA((n_tids,)),
            pltpu.SemaphoreType.DMA((n_tids,)),
            pltpu.SemaphoreType.REGULAR((n_tids,)),
            pltpu.SemaphoreType.DMA((3 * n_flows,)),
        ],
        compiler_params=pltpu.CompilerParams(
            collective_id=0,
            vmem_limit_bytes=56 * 1024 * 1024,
        ),
    )(p)


# device time: 844463 ns/iter; 1.8801x vs baseline; 1.0783x over previous
import jax
import jax.numpy as jnp
from jax import lax
from jax.experimental import pallas as pl
from jax.experimental.pallas import tpu as pltpu

M = 4096
N = 4096
KC = 1024

STREAMS = (
    (("x", "y", "z1", "z2"), 2560),
    (("y", "z1", "z2", "x"), 1536),
)
N_CHUNKS = 2
HALF = (2048, 1024, 512, 256)


def kernel(dy, W):
    my_x = lax.axis_index("x")
    my_z = lax.axis_index("z")
    idx8 = my_x * 4 + my_z
    dy_c = lax.dynamic_slice(dy, (0, idx8 * KC), (M, KC))
    w_c = lax.dynamic_slice(W, (0, idx8 * KC), (N, KC))
    p = lax.dot_general(
        dy_c, w_c, (((1,), (1,)), ((), ())),
        preferred_element_type=jnp.float32,
    )

    n_flows = len(STREAMS) * N_CHUNKS
    n_tids = 9 * n_flows

    def body(p_ref, o_ref, acc, recv, send_sems, recv_sems, ready, lsems):
        x = lax.axis_index("x")
        y = lax.axis_index("y")
        z = lax.axis_index("z")
        zb0 = z % 2
        zb1 = z // 2

        B = {"x": x, "y": y, "z1": zb0, "z2": zb1}
        P = {
            "x": (1 - x, y, z),
            "y": (x, 1 - y, z),
            "z1": (x, y, z + 1 - 2 * zb0),
            "z2": (x, y, z + 2 - 4 * zb1),
        }

        bar = pltpu.get_barrier_semaphore()
        for pid in P.values():
            pl.semaphore_signal(
                bar, inc=1, device_id=pid,
                device_id_type=pl.DeviceIdType.MESH,
            )
        pl.semaphore_wait(bar, 4)

        flows = []
        fi = 0
        col = 0
        for order, width in STREAMS:
            cw = width // N_CHUNKS
            for c in range(N_CHUNKS):
                f = {"order": order, "c0": col, "cw": cw, "fi": fi, "ci": c}
                fi += 1
                col += cw
                gb = 0
                gbs = [0]
                send_g = []
                for j, d in enumerate(order):
                    send_g.append(gb + (1 - B[d]) * HALF[j])
                    gb = gb + B[d] * HALF[j]
                    gbs.append(gb)
                f["gbs"] = gbs
                f["send_g"] = send_g
                flows.append(f)

        pend = {}

        def credit_and_start(tid, partner, src, dst):
            pl.semaphore_signal(
                ready.at[tid], inc=1, device_id=partner,
                device_id_type=pl.DeviceIdType.MESH,
            )
            pl.semaphore_wait(ready.at[tid], 1)
            rd = pltpu.make_async_remote_copy(
                src_ref=src, dst_ref=dst,
                send_sem=send_sems.at[tid], recv_sem=recv_sems.at[tid],
                device_id=partner, device_id_type=pl.DeviceIdType.MESH,
            )
            rd.start()
            return rd

        def issue(f, w):
            fi, c0, cw = f["fi"], f["c0"], f["cw"]
            gbs, order = f["gbs"], f["order"]
            cols = pl.ds(c0, cw)
            tid = 9 * fi + w
            if w in (0, 1):
                sub = w
                if w == 0:
                    f["cps"] = []
                    for s in range(2):
                        cp = pltpu.make_async_copy(
                            p_ref.at[pl.ds(gbs[1] + s * 1024, 1024), cols],
                            acc.at[pl.ds(s * 1024, 1024), cols],
                            lsems.at[3 * fi + s],
                        )
                        cp.start()
                        f["cps"].append(cp)
                rd = credit_and_start(
                    tid, P[order[0]],
                    p_ref.at[pl.ds(f["send_g"][0] + sub * 1024, 1024), cols],
                    recv.at[pl.ds(0, 1024), cols],
                )

                def fin(rd=rd, sub=sub, cp=f["cps"][sub], c0=c0, cw=cw):
                    rd.wait()
                    cp.wait()
                    r = pl.ds(sub * 1024, 1024)
                    acc[r, pl.ds(c0, cw)] = (
                        acc[r, pl.ds(c0, cw)] + recv[0:1024, c0:c0 + cw])
                pend[(fi, w)] = fin
            elif w in (2, 3, 4):
                j = w - 1
                n = HALF[j]
                cb_src = f["send_g"][j] - gbs[1]
                cb_dst = gbs[j + 1] - gbs[1]
                rd = credit_and_start(
                    tid, P[order[j]],
                    acc.at[pl.ds(cb_src, n), cols],
                    recv.at[pl.ds(0, n), cols],
                )

                def fin(rd=rd, n=n, cb_dst=cb_dst, c0=c0, cw=cw):
                    rd.wait()
                    r = pl.ds(cb_dst, n)
                    acc[r, pl.ds(c0, cw)] = (
                        acc[r, pl.ds(c0, cw)] + recv[0:n, c0:c0 + cw])
                pend[(fi, w)] = fin
            elif w == 5:
                cp = pltpu.make_async_copy(
                    acc.at[pl.ds(gbs[4] - gbs[1], 256), cols],
                    o_ref.at[pl.ds(gbs[4], 256), cols],
                    lsems.at[3 * fi + 2],
                )
                cp.start()
                rd = credit_and_start(
                    tid, P[order[3]],
                    acc.at[pl.ds(gbs[4] - gbs[1], 256), cols],
                    o_ref.at[pl.ds(gbs[4], 256), cols],
                )

                def fin(rd=rd, cp=cp):
                    rd.wait()
                    cp.wait()
                pend[(fi, w)] = fin
            else:
                k = w - 5
                jlev = 4 - k
                n = HALF[jlev - 1]
                rd = credit_and_start(
                    tid, P[order[jlev - 1]],
                    o_ref.at[pl.ds(gbs[jlev], n), cols],
                    o_ref.at[pl.ds(gbs[jlev], n), cols],
                )

                def fin(rd=rd):
                    rd.wait()
                pend[(fi, w)] = fin

        for t in range(9 + N_CHUNKS - 1):
            for f in flows:
                w = t - f["ci"]
                if 0 <= w <= 8:
                    if w > 0:
                        pend.pop((f["fi"], w - 1))()
                    issue(f, w)
        for f in flows:
            pend.pop((f["fi"], 8))()

    return pl.pallas_call(
        body,
        out_shape=jax.ShapeDtypeStruct((M, N), jnp.float32),
        in_specs=[pl.BlockSpec(memory_space=pl.ANY)],
        out_specs=pl.BlockSpec(memory_space=pl.ANY),
        scratch_shapes=[
            pltpu.VMEM((2048, N), jnp.float32),
            pltpu.VMEM((1024, N), jnp.float32),
            pltpu.SemaphoreType.DMA((n_tids,)),
            pltpu.SemaphoreType.DMA((n_tids,)),
            pltpu.SemaphoreType.REGULAR((n_tids,)),
            pltpu.SemaphoreType.DMA((3 * n_flows,)),
        ],
        compiler_params=pltpu.CompilerParams(
            collective_id=0,
            vmem_limit_bytes=56 * 1024 * 1024,
        ),
    )(p)


# device time: 724283 ns/iter; 2.1921x vs baseline; 1.1659x over previous
import jax
import jax.numpy as jnp
from jax import lax
from jax.experimental import pallas as pl
from jax.experimental.pallas import tpu as pltpu

M = 4096
N = 4096
KC = 1024

STREAMS = (
    (("x", "y", "z1", "z2"), 2560),
    (("y", "z1", "z2", "x"), 1536),
)
N_CHUNKS = 4
HALF = (2048, 1024, 512, 256)


def kernel(dy, W):
    my_x = lax.axis_index("x")
    my_z = lax.axis_index("z")
    idx8 = my_x * 4 + my_z
    dy_c = lax.dynamic_slice(dy, (0, idx8 * KC), (M, KC))
    w_c = lax.dynamic_slice(W, (0, idx8 * KC), (N, KC))
    p = lax.dot_general(
        dy_c, w_c, (((1,), (1,)), ((), ())),
        preferred_element_type=jnp.float32,
    )

    n_flows = len(STREAMS) * N_CHUNKS
    n_tids = 9 * n_flows

    def body(p_ref, o_ref, acc, recv, send_sems, recv_sems, ready, lsems):
        x = lax.axis_index("x")
        y = lax.axis_index("y")
        z = lax.axis_index("z")
        zb0 = z % 2
        zb1 = z // 2

        B = {"x": x, "y": y, "z1": zb0, "z2": zb1}
        P = {
            "x": (1 - x, y, z),
            "y": (x, 1 - y, z),
            "z1": (x, y, z + 1 - 2 * zb0),
            "z2": (x, y, z + 2 - 4 * zb1),
        }

        bar = pltpu.get_barrier_semaphore()
        for pid in P.values():
            pl.semaphore_signal(
                bar, inc=1, device_id=pid,
                device_id_type=pl.DeviceIdType.MESH,
            )
        pl.semaphore_wait(bar, 4)

        flows = []
        fi = 0
        col = 0
        for order, width in STREAMS:
            cw = width // N_CHUNKS
            for c in range(N_CHUNKS):
                f = {"order": order, "c0": col, "cw": cw, "fi": fi, "ci": c}
                fi += 1
                col += cw
                gb = 0
                gbs = [0]
                send_g = []
                for j, d in enumerate(order):
                    send_g.append(gb + (1 - B[d]) * HALF[j])
                    gb = gb + B[d] * HALF[j]
                    gbs.append(gb)
                f["gbs"] = gbs
                f["send_g"] = send_g
                flows.append(f)

        pend = {}

        def credit_and_start(tid, partner, src, dst):
            pl.semaphore_signal(
                ready.at[tid], inc=1, device_id=partner,
                device_id_type=pl.DeviceIdType.MESH,
            )
            pl.semaphore_wait(ready.at[tid], 1)
            rd = pltpu.make_async_remote_copy(
                src_ref=src, dst_ref=dst,
                send_sem=send_sems.at[tid], recv_sem=recv_sems.at[tid],
                device_id=partner, device_id_type=pl.DeviceIdType.MESH,
            )
            rd.start()
            return rd

        def issue(f, w):
            fi, c0, cw = f["fi"], f["c0"], f["cw"]
            gbs, order = f["gbs"], f["order"]
            cols = pl.ds(c0, cw)
            tid = 9 * fi + w
            if w in (0, 1):
                sub = w
                if w == 0:
                    f["cps"] = []
                    for s in range(2):
                        cp = pltpu.make_async_copy(
                            p_ref.at[pl.ds(gbs[1] + s * 1024, 1024), cols],
                            acc.at[pl.ds(s * 1024, 1024), cols],
                            lsems.at[3 * fi + s],
                        )
                        cp.start()
                        f["cps"].append(cp)
                rd = credit_and_start(
                    tid, P[order[0]],
                    p_ref.at[pl.ds(f["send_g"][0] + sub * 1024, 1024), cols],
                    recv.at[pl.ds(0, 1024), cols],
                )

                def fin(rd=rd, sub=sub, cp=f["cps"][sub], c0=c0, cw=cw):
                    rd.wait()
                    cp.wait()
                    r = pl.ds(sub * 1024, 1024)
                    acc[r, pl.ds(c0, cw)] = (
                        acc[r, pl.ds(c0, cw)] + recv[0:1024, c0:c0 + cw])
                pend[(fi, w)] = fin
            elif w in (2, 3, 4):
                j = w - 1
                n = HALF[j]
                cb_src = f["send_g"][j] - gbs[1]
                cb_dst = gbs[j + 1] - gbs[1]
                rd = credit_and_start(
                    tid, P[order[j]],
                    acc.at[pl.ds(cb_src, n), cols],
                    recv.at[pl.ds(0, n), cols],
                )

                def fin(rd=rd, n=n, cb_dst=cb_dst, c0=c0, cw=cw):
                    rd.wait()
                    r = pl.ds(cb_dst, n)
                    acc[r, pl.ds(c0, cw)] = (
                        acc[r, pl.ds(c0, cw)] + recv[0:n, c0:c0 + cw])
                pend[(fi, w)] = fin
            elif w == 5:
                cp = pltpu.make_async_copy(
                    acc.at[pl.ds(gbs[4] - gbs[1], 256), cols],
                    o_ref.at[pl.ds(gbs[4], 256), cols],
                    lsems.at[3 * fi + 2],
                )
                cp.start()
                rd = credit_and_start(
                    tid, P[order[3]],
                    acc.at[pl.ds(gbs[4] - gbs[1], 256), cols],
                    o_ref.at[pl.ds(gbs[4], 256), cols],
                )

                def fin(rd=rd, cp=cp):
                    rd.wait()
                    cp.wait()
                pend[(fi, w)] = fin
            else:
                k = w - 5
                jlev = 4 - k
                n = HALF[jlev - 1]
                rd = credit_and_start(
                    tid, P[order[jlev - 1]],
                    o_ref.at[pl.ds(gbs[jlev], n), cols],
                    o_ref.at[pl.ds(gbs[jlev], n), cols],
                )

                def fin(rd=rd):
                    rd.wait()
                pend[(fi, w)] = fin

        for t in range(9 + N_CHUNKS - 1):
            for f in flows:
                w = t - f["ci"]
                if 0 <= w <= 8:
                    if w > 0:
                        pend.pop((f["fi"], w - 1))()
                    issue(f, w)
        for f in flows:
            pend.pop((f["fi"], 8))()

    return pl.pallas_call(
        body,
        out_shape=jax.ShapeDtypeStruct((M, N), jnp.float32),
        in_specs=[pl.BlockSpec(memory_space=pl.ANY)],
        out_specs=pl.BlockSpec(memory_space=pl.ANY),
        scratch_shapes=[
            pltpu.VMEM((2048, N), jnp.float32),
            pltpu.VMEM((1024, N), jnp.float32),
            pltpu.SemaphoreType.DMA((n_tids,)),
            pltpu.SemaphoreType.DMA((n_tids,)),
            pltpu.SemaphoreType.REGULAR((n_tids,)),
            pltpu.SemaphoreType.DMA((3 * n_flows,)),
        ],
        compiler_params=pltpu.CompilerParams(
            collective_id=0,
            vmem_limit_bytes=56 * 1024 * 1024,
        ),
    )(p)
